# baseline (speedup 1.0000x reference)
import math

import jax
import jax.numpy as jnp
from jax import lax
from jax.experimental import pallas as pl
from jax.experimental.pallas import tpu as pltpu

_VMEM_LIMIT_BYTES = 64 * 1024 * 1024


def _mm_bf16_kernel(x_ref, w_ref, o_ref, wbf_ref):
    @pl.when(pl.program_id(1) == 0)
    def _():
        wbf_ref[...] = w_ref[...].astype(jnp.bfloat16)

    o_ref[...] = lax.dot_general(
        x_ref[...].astype(jnp.bfloat16),
        wbf_ref[...],
        dimension_numbers=(((1,), (1,)), ((), ())),
        preferred_element_type=jnp.float32,
    )


@jax.jit
def kernel(x, weight):
    dim = x.shape[-1]
    lead = x.shape[:-1]
    m = math.prod(lead) if lead else 1
    x2d = x.reshape(m, dim)

    block_m = min(m, 2048)
    n_blocks = pl.cdiv(m, block_m)
    if n_blocks % 2 == 0:
        grid = (2, n_blocks // 2)
    else:
        grid = (1, n_blocks)

    out2d = pl.pallas_call(
        _mm_bf16_kernel,
        out_shape=jax.ShapeDtypeStruct((m, dim), x.dtype),
        grid=grid,
        in_specs=[
            pl.BlockSpec((block_m, dim), lambda i, j: (i * grid[1] + j, 0)),
            pl.BlockSpec((dim, dim), lambda i, j: (0, 0)),
        ],
        out_specs=pl.BlockSpec((block_m, dim), lambda i, j: (i * grid[1] + j, 0)),
        scratch_shapes=[pltpu.VMEM((dim, dim), jnp.bfloat16)],
        compiler_params=pltpu.CompilerParams(
            dimension_semantics=("parallel", "arbitrary"),
            vmem_limit_bytes=_VMEM_LIMIT_BYTES,
        ),
    )(x2d, weight)
    return out2d.reshape(*lead, dim)


# speedup vs baseline: 1.0164x; 1.0164x over previous
import math

import jax
import jax.numpy as jnp
from jax import lax
from jax.experimental import pallas as pl
from jax.experimental.pallas import tpu as pltpu

_VMEM_LIMIT_BYTES = 64 * 1024 * 1024


def _mm_bf16_kernel(x_ref, w_ref, o_ref):
    o_ref[...] = lax.dot_general(
        x_ref[...].astype(jnp.bfloat16),
        w_ref[...].astype(jnp.bfloat16),
        dimension_numbers=(((1,), (1,)), ((), ())),
        preferred_element_type=jnp.float32,
    )


@jax.jit
def kernel(x, weight):
    dim = x.shape[-1]
    lead = x.shape[:-1]
    m = math.prod(lead) if lead else 1
    x2d = x.reshape(m, dim)

    block_m = min(m, 2048)
    grid = (pl.cdiv(m, block_m),)

    out2d = pl.pallas_call(
        _mm_bf16_kernel,
        out_shape=jax.ShapeDtypeStruct((m, dim), x.dtype),
        grid=grid,
        in_specs=[
            pl.BlockSpec((block_m, dim), lambda i: (i, 0)),
            pl.BlockSpec((dim, dim), lambda i: (0, 0)),
        ],
        out_specs=pl.BlockSpec((block_m, dim), lambda i: (i, 0)),
        compiler_params=pltpu.CompilerParams(
            dimension_semantics=("parallel",),
            vmem_limit_bytes=_VMEM_LIMIT_BYTES,
        ),
    )(x2d, weight)
    return out2d.reshape(*lead, dim)
